# baseline (speedup 1.0000x reference)
import jax
import jax.numpy as jnp
from jax import lax
from jax.experimental import pallas as pl
from jax.experimental.pallas import tpu as pltpu
from jax.experimental.pallas import tpu_sc as plsc

_N = 16384
_M = 200
_D = 64
_DP = 128
_V = 1000
_NC, _NS = 2, 16
_NW = _NC * _NS
_ROWS_PER_W = _N // _NW
_K = 2
_G = _ROWS_PER_W // _K
_CH = ((0, 128), (128, _M - 128))


def _emb_body(idx_hbm, table_hbm, out_hbm, idx_v, rows_v, gsem, osem):
    sid = lax.axis_index("s")
    wid = sid * _NC + lax.axis_index("c")
    row_base = wid * _ROWS_PER_W

    def fire_group(g, b):
        r0 = row_base + g * _K
        pltpu.sync_copy(idx_hbm.at[pl.ds(r0, _K)], idx_v.at[b])
        for j in range(_K):
            for (o, w) in _CH:
                pltpu.async_copy(
                    table_hbm.at[idx_v.at[b].at[j].at[pl.ds(o, w)]],
                    rows_v.at[b].at[j].at[pl.ds(o, w)],
                    gsem,
                )

    def drain_group(b):
        for j in range(_K):
            for (o, w) in _CH:
                pltpu.make_async_copy(
                    table_hbm.at[idx_v.at[b].at[j].at[pl.ds(o, w)]],
                    rows_v.at[b].at[j].at[pl.ds(o, w)],
                    gsem,
                ).wait()

    def drain_out(b):
        pltpu.make_async_copy(
            rows_v.at[b], out_hbm.at[pl.ds(0, _K)], osem
        ).wait()

    fire_group(0, 0)

    def pair(p, carry):
        g0 = p * 2
        for b in range(2):
            gg = g0 + b
            nb = 1 - b
            drain_group(b)
            pltpu.async_copy(
                rows_v.at[b], out_hbm.at[pl.ds(row_base + gg * _K, _K)], osem
            )

            @pl.when(gg >= 1)
            def _():
                drain_out(nb)

            @pl.when(gg + 1 < _G)
            def _():
                fire_group(gg + 1, nb)
        return carry

    lax.fori_loop(0, _G // 2, pair, 0)
    drain_out(1)


def kernel(input, weight):
    wp = jnp.pad(weight, ((0, 0), (0, _DP - _D)))
    mesh = plsc.VectorSubcoreMesh(core_axis_name="c", subcore_axis_name="s")
    call = pl.kernel(
        _emb_body,
        out_type=jax.ShapeDtypeStruct((_N, _M, _DP), jnp.float32),
        mesh=mesh,
        scratch_types=[
            pltpu.VMEM((2, _K, _M), jnp.int32),
            pltpu.VMEM((2, _K, _M, _DP), jnp.float32),
            pltpu.SemaphoreType.DMA,
            pltpu.SemaphoreType.DMA,
        ],
        compiler_params=pltpu.CompilerParams(use_tc_tiling_on_sc=True),
    )
    out = call(input.astype(jnp.int32), wp)
    return out[:, :, :_D]


# speedup vs baseline: 1.0966x; 1.0966x over previous
import jax
import jax.numpy as jnp
from jax import lax
from jax.experimental import pallas as pl
from jax.experimental.pallas import tpu as pltpu
from jax.experimental.pallas import tpu_sc as plsc

_N = 16384
_M = 200
_D = 64
_V = 1000
_NC, _NS = 2, 16
_NW = _NC * _NS
_ROWS_PER_W = _N // _NW
_K = 4
_G = _ROWS_PER_W // _K
_CH = ((0, 128), (128, _M - 128))


def _emb_body(idx_hbm, table_hbm, out_hbm, table_sh, idx_v, rows_v, gsem, osem):
    sid = lax.axis_index("s")
    wid = sid * _NC + lax.axis_index("c")
    row_base = wid * _ROWS_PER_W

    pltpu.sync_copy(table_hbm, table_sh)

    def fire_group(g, b):
        r0 = row_base + g * _K
        pltpu.sync_copy(idx_hbm.at[pl.ds(r0, _K)], idx_v.at[b])
        for j in range(_K):
            for (o, w) in _CH:
                pltpu.async_copy(
                    table_sh.at[idx_v.at[b].at[j].at[pl.ds(o, w)]],
                    rows_v.at[b].at[j].at[pl.ds(o, w)],
                    gsem,
                )

    def drain_group(b):
        for j in range(_K):
            for (o, w) in _CH:
                pltpu.make_async_copy(
                    table_sh.at[idx_v.at[b].at[j].at[pl.ds(o, w)]],
                    rows_v.at[b].at[j].at[pl.ds(o, w)],
                    gsem,
                ).wait()

    def drain_out(b):
        pltpu.make_async_copy(
            rows_v.at[b], out_hbm.at[pl.ds(0, _K)], osem
        ).wait()

    fire_group(0, 0)

    def pair(p, carry):
        g0 = p * 2
        for b in range(2):
            gg = g0 + b
            nb = 1 - b
            drain_group(b)
            pltpu.async_copy(
                rows_v.at[b], out_hbm.at[pl.ds(row_base + gg * _K, _K)], osem
            )

            @pl.when(gg >= 1)
            def _():
                drain_out(nb)

            @pl.when(gg + 1 < _G)
            def _():
                fire_group(gg + 1, nb)
        return carry

    lax.fori_loop(0, _G // 2, pair, 0)
    drain_out(1)


def kernel(input, weight):
    mesh = plsc.VectorSubcoreMesh(core_axis_name="c", subcore_axis_name="s")
    call = pl.kernel(
        _emb_body,
        out_type=jax.ShapeDtypeStruct((_N, _M, _D), jnp.float32),
        mesh=mesh,
        scratch_types=[
            pltpu.VMEM_SHARED((_V, _D), jnp.float32),
            pltpu.VMEM((2, _K, _M), jnp.int32),
            pltpu.VMEM((2, _K, _M, _D), jnp.float32),
            pltpu.SemaphoreType.DMA,
            pltpu.SemaphoreType.DMA,
        ],
        compiler_params=pltpu.CompilerParams(use_tc_tiling_on_sc=False),
    )
    return call(input.astype(jnp.int32), weight)
